# baseline (speedup 1.0000x reference)
import jax
import jax.numpy as jnp
from jax.experimental import pallas as pl

_N_EXP = 16
_TOP_K = 2


def _router_kernel(xf_ref, gw_ref, gb_ref, comb_ref):
    scores = jnp.dot(xf_ref[...], gw_ref[...],
                     preferred_element_type=jnp.float32) + gb_ref[...]
    col = jax.lax.broadcasted_iota(jnp.int32, scores.shape, 1)
    m1 = jnp.max(scores, axis=1, keepdims=True)
    idx1 = jnp.min(jnp.where(scores == m1, col, _N_EXP), axis=1, keepdims=True)
    masked = jnp.where(col == idx1, -jnp.inf, scores)
    m2 = jnp.max(masked, axis=1, keepdims=True)
    idx2 = jnp.min(jnp.where(masked == m2, col, _N_EXP), axis=1, keepdims=True)
    comb_ref[...] = (jnp.where(col == idx1, m1, 0.0)
                     + jnp.where(col == idx2, m2, 0.0))


def _moe_kernel(comb_ref, xf_ref, wup_ref, bup_ref, wg_ref, bg_ref,
                wdn_ref, bdn_ref, out_ref):
    e = pl.program_id(0)
    f = pl.program_id(1)

    @pl.when((e == 0) & (f == 0))
    def _init():
        out_ref[...] = jnp.zeros_like(out_ref)

    x = xf_ref[...]
    u = jnp.dot(x, wup_ref[0], preferred_element_type=jnp.float32) + bup_ref[0, 0]
    g = jnp.dot(x, wg_ref[0], preferred_element_type=jnp.float32) + bg_ref[0, 0]
    h = u * (g * jax.lax.logistic(g))
    part = jnp.dot(h, wdn_ref[0], preferred_element_type=jnp.float32)

    comb = comb_ref[...]
    col = jax.lax.broadcasted_iota(jnp.int32, comb.shape, 1)
    c = jnp.sum(jnp.where(col == e, comb, 0.0), axis=1, keepdims=True)

    @pl.when(f == 0)
    def _bias():
        out_ref[...] += c * bdn_ref[0]

    out_ref[...] += c * part


def kernel(x, gate_W, gate_b, W_up, b_up, W_g, b_g, W_down, b_down):
    B, T, C = x.shape
    D_FF = W_up.shape[2]
    F = 128
    NF = D_FF // F
    xf = x.reshape(T, C)

    comb = pl.pallas_call(
        _router_kernel,
        out_shape=jax.ShapeDtypeStruct((T, _N_EXP), jnp.float32),
    )(xf, gate_W, gate_b.reshape(1, _N_EXP))

    out = pl.pallas_call(
        _moe_kernel,
        grid=(_N_EXP, NF),
        in_specs=[
            pl.BlockSpec((T, _N_EXP), lambda e, f: (0, 0)),
            pl.BlockSpec((T, C), lambda e, f: (0, 0)),
            pl.BlockSpec((1, C, F), lambda e, f: (e, 0, f)),
            pl.BlockSpec((1, 1, 1, F), lambda e, f: (e, f, 0, 0)),
            pl.BlockSpec((1, C, F), lambda e, f: (e, 0, f)),
            pl.BlockSpec((1, 1, 1, F), lambda e, f: (e, f, 0, 0)),
            pl.BlockSpec((1, F, C), lambda e, f: (e, f, 0)),
            pl.BlockSpec((1, 1, C), lambda e, f: (e, 0, 0)),
        ],
        out_specs=pl.BlockSpec((T, C), lambda e, f: (0, 0)),
        out_shape=jax.ShapeDtypeStruct((T, C), jnp.float32),
    )(comb, xf, W_up, b_up.reshape(_N_EXP, NF, 1, F), W_g,
      b_g.reshape(_N_EXP, NF, 1, F), W_down, b_down.reshape(_N_EXP, 1, C))

    return out.reshape(B, T, C)


# speedup vs baseline: 2.4251x; 2.4251x over previous
import jax
import jax.numpy as jnp
from jax.experimental import pallas as pl
from jax.experimental.pallas import tpu as pltpu

_N_EXP = 16
_TOP_K = 2
_M = 256
_F = 128


def _router_kernel(xf_ref, gw_ref, gb_ref, idx_ref, tv_ref):
    scores = jnp.dot(xf_ref[...], gw_ref[...],
                     preferred_element_type=jnp.float32) + gb_ref[...]
    col = jax.lax.broadcasted_iota(jnp.int32, scores.shape, 1)
    m1 = jnp.max(scores, axis=1, keepdims=True)
    idx1 = jnp.min(jnp.where(scores == m1, col, _N_EXP), axis=1, keepdims=True)
    masked = jnp.where(col == idx1, -jnp.inf, scores)
    m2 = jnp.max(masked, axis=1, keepdims=True)
    idx2 = jnp.min(jnp.where(masked == m2, col, _N_EXP), axis=1, keepdims=True)
    idx_ref[...] = jnp.concatenate([idx1, idx2], axis=1)
    tv_ref[...] = jnp.concatenate([m1, m2], axis=1)


def _plan_kernel(idx_ref, tv_ref, cnt_ref, cblk_ref, off_ref, tok_ref,
                 tvs_ref, next_ref):
    T = idx_ref.shape[0] // _TOP_K

    def zero(e, c):
        cnt_ref[e] = 0
        return c

    jax.lax.fori_loop(0, _N_EXP, zero, 0)

    def count(t, c):
        e0 = idx_ref[2 * t]
        e1 = idx_ref[2 * t + 1]
        cnt_ref[e0] = cnt_ref[e0] + 1
        cnt_ref[e1] = cnt_ref[e1] + 1
        return c

    jax.lax.fori_loop(0, T, count, 0)

    def offs(e, a):
        ce = cnt_ref[e]
        off_ref[e] = a
        next_ref[e] = a
        cblk_ref[e] = (ce + _M - 1) // _M
        return a + ce

    jax.lax.fori_loop(0, _N_EXP, offs, 0)

    def scatter(t, c):
        e0 = idx_ref[2 * t]
        s0 = next_ref[e0]
        tok_ref[s0] = t
        tvs_ref[s0] = tv_ref[2 * t]
        next_ref[e0] = s0 + 1
        e1 = idx_ref[2 * t + 1]
        s1 = next_ref[e1]
        tok_ref[s1] = t
        tvs_ref[s1] = tv_ref[2 * t + 1]
        next_ref[e1] = s1 + 1
        return c

    jax.lax.fori_loop(0, T, scatter, 0)


def _moe_kernel(cnt_ref, cblk_ref, off_ref, tok_ref, tvs_ref, xf_ref,
                wup_ref, bup_ref, wg_ref, bg_ref, wdn_ref, bdn_ref,
                out_ref, xblk_ref, acc_ref):
    e = pl.program_id(0)
    f = pl.program_id(1)
    nf = pl.num_programs(1)
    ce = cnt_ref[e]
    nb = cblk_ref[e]
    o = off_ref[e]

    @pl.when((e == 0) & (f == 0))
    def _init():
        out_ref[...] = jnp.zeros_like(out_ref)

    @pl.when(f == 0)
    def _gather():
        def gath(r, c):
            t = tok_ref[o + r]
            xblk_ref[pl.ds(r, 1), :] = xf_ref[pl.ds(t, 1), :]
            return c

        jax.lax.fori_loop(0, ce, gath, 0)

    bup = bup_ref[0, 0]
    bg = bg_ref[0, 0]
    bdn = bdn_ref[0]
    isf0 = f == 0

    def comp(rb, c):
        xb = xblk_ref[pl.ds(rb * _M, _M), :]
        u = jnp.dot(xb, wup_ref[0], preferred_element_type=jnp.float32) + bup
        g = jnp.dot(xb, wg_ref[0], preferred_element_type=jnp.float32) + bg
        h = u * (g * jax.lax.logistic(g))
        p = jnp.dot(h, wdn_ref[0], preferred_element_type=jnp.float32)
        prev = acc_ref[pl.ds(rb * _M, _M), :]
        acc_ref[pl.ds(rb * _M, _M), :] = p + jnp.where(
            isf0, jnp.broadcast_to(bdn, p.shape), prev)
        return c

    jax.lax.fori_loop(0, nb, comp, 0)

    @pl.when(f == nf - 1)
    def _scatter():
        def scat(r, c):
            t = tok_ref[o + r]
            w = tvs_ref[o + r]
            out_ref[pl.ds(t, 1), :] += w * acc_ref[pl.ds(r, 1), :]
            return c

        jax.lax.fori_loop(0, ce, scat, 0)


def kernel(x, gate_W, gate_b, W_up, b_up, W_g, b_g, W_down, b_down):
    B, T, C = x.shape
    D_FF = W_up.shape[2]
    NF = D_FF // _F
    S = _TOP_K * T
    TP = ((T + _M - 1) // _M) * _M
    xf = x.reshape(T, C)

    idx, tv = pl.pallas_call(
        _router_kernel,
        out_shape=(jax.ShapeDtypeStruct((T, _TOP_K), jnp.int32),
                   jax.ShapeDtypeStruct((T, _TOP_K), jnp.float32)),
    )(xf, gate_W, gate_b.reshape(1, _N_EXP))

    smem = pl.BlockSpec(memory_space=pltpu.SMEM)
    cnt, cblk, off, tok, tvs = pl.pallas_call(
        _plan_kernel,
        in_specs=[smem, smem],
        out_specs=(smem, smem, smem, smem, smem),
        out_shape=(jax.ShapeDtypeStruct((_N_EXP,), jnp.int32),
                   jax.ShapeDtypeStruct((_N_EXP,), jnp.int32),
                   jax.ShapeDtypeStruct((_N_EXP,), jnp.int32),
                   jax.ShapeDtypeStruct((S,), jnp.int32),
                   jax.ShapeDtypeStruct((S,), jnp.float32)),
        scratch_shapes=[pltpu.SMEM((_N_EXP,), jnp.int32)],
    )(idx.reshape(-1), tv.reshape(-1))

    out = pl.pallas_call(
        _moe_kernel,
        grid=(_N_EXP, NF),
        in_specs=[
            smem, smem, smem, smem, smem,
            pl.BlockSpec((T, C), lambda e, f: (0, 0)),
            pl.BlockSpec((1, C, _F), lambda e, f: (e, 0, f)),
            pl.BlockSpec((1, 1, 1, _F), lambda e, f: (e, f, 0, 0)),
            pl.BlockSpec((1, C, _F), lambda e, f: (e, 0, f)),
            pl.BlockSpec((1, 1, 1, _F), lambda e, f: (e, f, 0, 0)),
            pl.BlockSpec((1, _F, C), lambda e, f: (e, f, 0)),
            pl.BlockSpec((1, 1, C), lambda e, f: (e, 0, 0)),
        ],
        out_specs=pl.BlockSpec((T, C), lambda e, f: (0, 0)),
        out_shape=jax.ShapeDtypeStruct((T, C), jnp.float32),
        scratch_shapes=[pltpu.VMEM((TP, C), jnp.float32),
                        pltpu.VMEM((TP, C), jnp.float32)],
    )(cnt, cblk, off, tok, tvs, xf, W_up, b_up.reshape(_N_EXP, NF, 1, _F),
      W_g, b_g.reshape(_N_EXP, NF, 1, _F), W_down,
      b_down.reshape(_N_EXP, 1, C))

    return out.reshape(B, T, C)
